# Initial kernel scaffold; baseline (speedup 1.0000x reference)
import jax
import jax.numpy as jnp
from jax.experimental import pallas as pl


def kernel(adj, temperature, u):
    raise NotImplementedError("write your pallas kernel here")


# speedup vs baseline: 1.0147x; 1.0147x over previous
import jax
import jax.numpy as jnp
from jax.experimental import pallas as pl
from jax.experimental.pallas import tpu as pltpu


def _gumbel_rows_kernel(n, rblk, tref, adjref, u0ref, u1ref, yref):
    pid = pl.program_id(0)
    inv_t = 1.0 / tref[0]
    p = adjref[...]
    w0 = u0ref[...]
    w1 = u1ref[...]
    eps = 1e-20
    g0 = -jnp.log(-jnp.log(w0 + eps) + eps)
    g1 = -jnp.log(-jnp.log(w1 + eps) + eps)
    a0 = p + g0
    a1 = jnp.abs(1.0 - p) + g1
    y = 1.0 / (1.0 + jnp.exp((a1 - a0) * inv_t))
    cols = jax.lax.broadcasted_iota(jnp.int32, (rblk, n), 1)
    rows = pid * rblk + jax.lax.broadcasted_iota(jnp.int32, (rblk, n), 0)
    yref[...] = jnp.where(cols > rows, y, jnp.where(cols == rows, 1.0, 0.0))


def _symmetrize_kernel(bb, aref, bref, oref):
    bi = pl.program_id(0)
    bj = pl.program_id(1)
    rows = bi * bb + jax.lax.broadcasted_iota(jnp.int32, (bb, bb), 0)
    cols = bj * bb + jax.lax.broadcasted_iota(jnp.int32, (bb, bb), 1)
    diag = jnp.where(rows == cols, 1.0, 0.0)
    oref[...] = aref[...] + bref[...].T - diag


def kernel(adj, temperature, u):
    n = adj.shape[0]
    rblk = 8 if n % 8 == 0 else 1
    bb = 512 if n % 512 == 0 else n

    t = jnp.asarray(temperature, jnp.float32).reshape(1)

    i_idx, j_idx = jnp.triu_indices(n, k=1)
    u0d = jnp.zeros((n, n), jnp.float32).at[i_idx, j_idx].set(u[:, 0])
    u1d = jnp.zeros((n, n), jnp.float32).at[i_idx, j_idx].set(u[:, 1])

    grid_a = n // rblk
    upper = pl.pallas_call(
        lambda *refs: _gumbel_rows_kernel(n, rblk, *refs),
        grid=(grid_a,),
        in_specs=[
            pl.BlockSpec(memory_space=pltpu.SMEM),
            pl.BlockSpec((rblk, n), lambda p: (p, 0)),
            pl.BlockSpec((rblk, n), lambda p: (p, 0)),
            pl.BlockSpec((rblk, n), lambda p: (p, 0)),
        ],
        out_specs=pl.BlockSpec((rblk, n), lambda p: (p, 0)),
        out_shape=jax.ShapeDtypeStruct((n, n), jnp.float32),
    )(t, adj, u0d, u1d)

    nb = n // bb
    out = pl.pallas_call(
        lambda *refs: _symmetrize_kernel(bb, *refs),
        grid=(nb, nb),
        in_specs=[
            pl.BlockSpec((bb, bb), lambda i, j: (i, j)),
            pl.BlockSpec((bb, bb), lambda i, j: (j, i)),
        ],
        out_specs=pl.BlockSpec((bb, bb), lambda i, j: (i, j)),
        out_shape=jax.ShapeDtypeStruct((n, n), jnp.float32),
    )(upper, upper)
    return out
